# baseline (speedup 1.0000x reference)
import functools

import jax
import jax.numpy as jnp
from jax import lax
from jax.experimental import pallas as pl
from jax.experimental.pallas import tpu as pltpu
from jax.experimental.pallas import tpu_sc as plsc

B = 16384
L = 64
D = 128
V = 100000
EPS = 1e-5

NC = 2
NS = 16
NW = NC * NS
ROWS_PER_W = B // NW
CB = 4
IDS_PER_CHUNK = CB * L
NGATHER = IDS_PER_CHUNK // 128
NSTEPS = ROWS_PER_W // CB
TOK_UNROLL = 8
NVREG = D // 16


IDROWS_PER_W = ROWS_PER_W * L // 128


def _pool_body(ids_hbm, table_hbm, out_hbm, idx_v, rows_v, sums_v,
               sem0, sem1, osem0, osem1):
    c = lax.axis_index("c")
    s = lax.axis_index("s")
    wid = s * NC + c
    base = wid * ROWS_PER_W
    sems = (sem0, sem1)
    osems = (osem0, osem1)
    pltpu.sync_copy(ids_hbm.at[pl.ds(wid * IDROWS_PER_W, IDROWS_PER_W)], idx_v)

    def fire(chunk, p):
        for t in range(NGATHER):
            pltpu.async_copy(table_hbm.at[idx_v.at[chunk * NGATHER + t]],
                             rows_v.at[p].at[pl.ds(t * 128, 128)], sems[p])

    def drain(p):
        for t in range(NGATHER):
            pltpu.make_async_copy(table_hbm.at[idx_v.at[0]],
                                  rows_v.at[p].at[pl.ds(t * 128, 128)],
                                  sems[p]).wait()

    def drain_out(q):
        pltpu.make_async_copy(sums_v.at[q], out_hbm.at[pl.ds(base, 8)],
                              osems[q]).wait()

    fire(0, 0)

    def step(G, carry):
        for q in range(2):
            g = 2 * G + q

            @pl.when(G > 0)
            def _():
                drain_out(q)

            for h in range(2):
                chunk = g * 2 + h

                @pl.when(chunk + 1 < NSTEPS)
                def _():
                    fire(chunk + 1, 1 - h)

                drain(h)
                for cb in range(CB):
                    def body(j, acc):
                        r0 = cb * L + j * TOK_UNROLL
                        out = list(acc)
                        for t in range(TOK_UNROLL):
                            for k in range(NVREG):
                                out[k] = out[k] + rows_v[h, r0 + t,
                                                         pl.ds(k * 16, 16)]
                        return tuple(out)
                    acc = lax.fori_loop(
                        0, L // TOK_UNROLL, body,
                        tuple(jnp.zeros((16,), jnp.float32)
                              for _ in range(NVREG)))
                    for k in range(NVREG):
                        sums_v[q, h * CB + cb, pl.ds(k * 16, 16)] = acc[k]
            pltpu.async_copy(sums_v.at[q], out_hbm.at[pl.ds(base + g * 8, 8)],
                             osems[q])
        return carry

    lax.fori_loop(0, NSTEPS // 4, step, 0)
    drain_out(0)
    drain_out(1)


@functools.partial(jax.jit, static_argnames=())
def _pool(ids2d, table):
    mesh = plsc.VectorSubcoreMesh(core_axis_name="c", subcore_axis_name="s",
                                  num_cores=NC, num_subcores=NS)
    return pl.kernel(
        _pool_body,
        out_type=jax.ShapeDtypeStruct((B, D), jnp.float32),
        mesh=mesh,
        scratch_types=[
            pltpu.VMEM((IDROWS_PER_W, 128), jnp.int32),
            pltpu.VMEM((2, IDS_PER_CHUNK, D), jnp.float32),
            pltpu.VMEM((2, 8, D), jnp.float32),
            pltpu.SemaphoreType.DMA,
            pltpu.SemaphoreType.DMA,
            pltpu.SemaphoreType.DMA,
            pltpu.SemaphoreType.DMA,
        ],
    )(ids2d, table)


def _ln_body(sums_ref, gamma_ref, beta_ref, out_ref):
    x = sums_ref[...] * (1.0 / L)
    mu = jnp.mean(x, axis=-1, keepdims=True)
    xc = x - mu
    var = jnp.mean(xc * xc, axis=-1, keepdims=True)
    out_ref[...] = xc * lax.rsqrt(var + EPS) * gamma_ref[...] + beta_ref[...]


def _ln(sums, gamma2d, beta2d):
    blk = 1024
    return pl.pallas_call(
        _ln_body,
        grid=(B // blk,),
        in_specs=[
            pl.BlockSpec((blk, D), lambda i: (i, 0)),
            pl.BlockSpec((1, D), lambda i: (0, 0)),
            pl.BlockSpec((1, D), lambda i: (0, 0)),
        ],
        out_specs=pl.BlockSpec((blk, D), lambda i: (i, 0)),
        out_shape=jax.ShapeDtypeStruct((B, D), jnp.float32),
    )(sums, gamma2d, beta2d)


def kernel(ids, table, gamma, beta):
    ids2d = ids.astype(jnp.int32).reshape(B * L // 128, 128)
    sums = _pool(ids2d, table)
    return _ln(sums, gamma.reshape(1, D), beta.reshape(1, D))


# speedup vs baseline: 1.0452x; 1.0452x over previous
import functools

import jax
import jax.numpy as jnp
from jax import lax
from jax.experimental import pallas as pl
from jax.experimental.pallas import tpu as pltpu
from jax.experimental.pallas import tpu_sc as plsc

B = 16384
L = 64
D = 128
V = 100000
EPS = 1e-5

NC = 2
NS = 16
NW = NC * NS
ROWS_PER_W = B // NW
CB = 4
IDS_PER_CHUNK = CB * L
NGATHER = IDS_PER_CHUNK // 128
NSTEPS = ROWS_PER_W // CB
TOK_UNROLL = 4
NVREG = D // 16


IDROWS_PER_W = ROWS_PER_W * L // 128


def _pool_body(ids_hbm, table_hbm, out_hbm, idx_v, rows_v, sums_v,
               sem0, sem1, osem0, osem1):
    c = lax.axis_index("c")
    s = lax.axis_index("s")
    wid = s * NC + c
    base = wid * ROWS_PER_W
    sems = (sem0, sem1)
    osems = (osem0, osem1)
    pltpu.sync_copy(ids_hbm.at[pl.ds(wid * IDROWS_PER_W, IDROWS_PER_W)], idx_v)

    def fire(chunk, p):
        for t in range(NGATHER):
            pltpu.async_copy(table_hbm.at[idx_v.at[chunk * NGATHER + t]],
                             rows_v.at[p].at[pl.ds(t * 128, 128)], sems[p])

    def drain(p):
        for t in range(NGATHER):
            pltpu.make_async_copy(table_hbm.at[idx_v.at[0]],
                                  rows_v.at[p].at[pl.ds(t * 128, 128)],
                                  sems[p]).wait()

    def drain_out(q):
        pltpu.make_async_copy(sums_v.at[q], out_hbm.at[pl.ds(base, 8)],
                              osems[q]).wait()

    fire(0, 0)

    def step(G, carry):
        for q in range(2):
            g = 2 * G + q

            @pl.when(G > 0)
            def _():
                drain_out(q)

            for h in range(2):
                chunk = g * 2 + h

                @pl.when(chunk + 1 < NSTEPS)
                def _():
                    fire(chunk + 1, 1 - h)

                drain(h)
                for cb in range(CB):
                    def body(j, acc):
                        r0 = cb * L + j * TOK_UNROLL
                        out = list(acc)
                        for t in range(TOK_UNROLL):
                            for k in range(NVREG):
                                out[k] = out[k] + rows_v[h, r0 + t,
                                                         pl.ds(k * 16, 16)]
                        return tuple(out)
                    acc = lax.fori_loop(
                        0, L // TOK_UNROLL, body,
                        tuple(jnp.zeros((16,), jnp.float32)
                              for _ in range(NVREG)))
                    for k in range(NVREG):
                        sums_v[q, h * CB + cb, pl.ds(k * 16, 16)] = acc[k]
            pltpu.async_copy(sums_v.at[q], out_hbm.at[pl.ds(base + g * 8, 8)],
                             osems[q])
        return carry

    lax.fori_loop(0, NSTEPS // 4, step, 0)
    drain_out(0)
    drain_out(1)


@functools.partial(jax.jit, static_argnames=())
def _pool(ids2d, table):
    mesh = plsc.VectorSubcoreMesh(core_axis_name="c", subcore_axis_name="s",
                                  num_cores=NC, num_subcores=NS)
    return pl.kernel(
        _pool_body,
        out_type=jax.ShapeDtypeStruct((B, D), jnp.float32),
        mesh=mesh,
        scratch_types=[
            pltpu.VMEM((IDROWS_PER_W, 128), jnp.int32),
            pltpu.VMEM((2, IDS_PER_CHUNK, D), jnp.float32),
            pltpu.VMEM((2, 8, D), jnp.float32),
            pltpu.SemaphoreType.DMA,
            pltpu.SemaphoreType.DMA,
            pltpu.SemaphoreType.DMA,
            pltpu.SemaphoreType.DMA,
        ],
    )(ids2d, table)


def _ln_body(sums_ref, gamma_ref, beta_ref, out_ref):
    x = sums_ref[...] * (1.0 / L)
    mu = jnp.mean(x, axis=-1, keepdims=True)
    xc = x - mu
    var = jnp.mean(xc * xc, axis=-1, keepdims=True)
    out_ref[...] = xc * lax.rsqrt(var + EPS) * gamma_ref[...] + beta_ref[...]


def _ln(sums, gamma2d, beta2d):
    blk = 1024
    return pl.pallas_call(
        _ln_body,
        grid=(B // blk,),
        in_specs=[
            pl.BlockSpec((blk, D), lambda i: (i, 0)),
            pl.BlockSpec((1, D), lambda i: (0, 0)),
            pl.BlockSpec((1, D), lambda i: (0, 0)),
        ],
        out_specs=pl.BlockSpec((blk, D), lambda i: (i, 0)),
        out_shape=jax.ShapeDtypeStruct((B, D), jnp.float32),
    )(sums, gamma2d, beta2d)


def kernel(ids, table, gamma, beta):
    ids2d = ids.astype(jnp.int32).reshape(B * L // 128, 128)
    sums = _pool(ids2d, table)
    return _ln(sums, gamma.reshape(1, D), beta.reshape(1, D))
